# baseline (speedup 1.0000x reference)
import functools

import jax
import jax.numpy as jnp
from jax import lax
from jax.experimental import pallas as pl
from jax.experimental.pallas import tpu as pltpu
from jax.experimental.pallas import tpu_sc as plsc

N = 10000
E = 160000
ER = E // 128
NC, NS = 2, 16
BM = 1000
F32 = jnp.float32


def _sc_mesh():
    return plsc.VectorSubcoreMesh(core_axis_name="c", subcore_axis_name="s")


@functools.partial(
    pl.kernel,
    out_type=[jax.ShapeDtypeStruct((N,), F32),
              jax.ShapeDtypeStruct((N,), F32)],
    mesh=_sc_mesh(),
    scratch_types=[
        pltpu.VMEM_SHARED((N + 8,), F32),
        pltpu.VMEM((1024,), F32),
        pltpu.VMEM((128,), F32),
        pltpu.VMEM((40, 128), jnp.int32),
        pltpu.VMEM((640,), F32),
    ],
)
def _deg(e2d_hbm, deg0_hbm, deg1_hbm, deg_sp, zbuf, ones, idxb, dbuf):
    c = lax.axis_index("c")
    s = lax.axis_index("s")
    zv = jnp.zeros((16,), F32)
    for off in range(0, 1024, 16):
        zbuf[pl.ds(off, 16)] = zv
    ov = jnp.ones((16,), F32)
    for off in range(0, 128, 16):
        ones[pl.ds(off, 16)] = ov

    @pl.when(s < 10)
    def _():
        pltpu.sync_copy(zbuf.at[pl.ds(0, 1000)],
                        deg_sp.at[pl.ds(s * 1000, 1000)])

    plsc.subcore_barrier()

    def scatter_rows(nr, base):
        pltpu.sync_copy(e2d_hbm.at[1, pl.ds(base, nr)], idxb.at[pl.ds(0, nr)])

        def eb(j, carry):
            pltpu.sync_copy(ones, deg_sp.at[idxb.at[j]], add=True)
            return carry

        lax.fori_loop(0, nr, eb, 0)

    scatter_rows(40, 640 * c + 40 * s)

    plsc.subcore_barrier()

    def writeout(dst):
        @pl.when(s < 15)
        def _():
            pltpu.sync_copy(deg_sp.at[pl.ds(s * 640, 640)], dbuf)
            pltpu.sync_copy(dbuf, dst.at[pl.ds(s * 640, 640)])

        @pl.when(s == 15)
        def _():
            pltpu.sync_copy(deg_sp.at[pl.ds(9600, 400)],
                            dbuf.at[pl.ds(0, 400)])
            pltpu.sync_copy(dbuf.at[pl.ds(0, 400)],
                            dst.at[pl.ds(9600, 400)])

    @pl.when(c == 0)
    def _():
        writeout(deg0_hbm)

    @pl.when(c == 1)
    def _():
        writeout(deg1_hbm)


def _make_agg(C):
    C2 = C // NC

    def body(e2d_hbm, g_hbm, s_hbm, acc, ridx, cidx, bufa, obuf, zbuf,
             sema, semb):
        c = lax.axis_index("c")
        s = lax.axis_index("s")

        zv = jnp.zeros((16,), F32)

        def zrow(r, carry):
            for off in range(0, 128, 16):
                zbuf[r, pl.ds(off, 16)] = zv
            return carry

        lax.fori_loop(0, 16, zrow, 0)

        nslabs = jnp.where(s == 0, 40, 39)
        nbase = 16 * (39 * s + jnp.minimum(s, 1))

        pltpu.sync_copy(e2d_hbm.at[0, pl.ds(s * 80, 80)], ridx)
        pltpu.sync_copy(e2d_hbm.at[1, pl.ds(s * 80, 80)], cidx)

        def chunk_pass(qi, carry):
            q = c * C2 + qi
            g = g_hbm.at[q]
            so = s_hbm.at[q]

            def zi(i, cy):
                pltpu.sync_copy(zbuf, acc.at[pl.ds(nbase + 16 * i, 16)])
                return cy

            lax.fori_loop(0, nslabs, zi, 0)

            plsc.subcore_barrier()

            def eb(j, cy):
                pltpu.async_copy(g.at[ridx.at[j]], bufa, sema).wait()
                pltpu.sync_copy(bufa, acc.at[cidx.at[j]], add=True)
                return cy

            lax.fori_loop(0, 80, eb, 0)

            plsc.subcore_barrier()

            def wi(i, cy):
                base = nbase + 16 * i
                pltpu.sync_copy(acc.at[pl.ds(base, 16)], obuf)
                pltpu.sync_copy(obuf, so.at[pl.ds(base, 16)])
                return cy

            lax.fori_loop(0, nslabs, wi, 0)

            plsc.subcore_barrier()
            return carry

        lax.fori_loop(0, C2, chunk_pass, 0)

    return pl.kernel(
        body,
        out_type=jax.ShapeDtypeStruct((C, N, 128), F32),
        mesh=_sc_mesh(),
        scratch_types=[
            pltpu.VMEM_SHARED((N + 8, 128), F32),
            pltpu.VMEM((80, 128), jnp.int32),
            pltpu.VMEM((80, 128), jnp.int32),
            pltpu.VMEM((128, 128), F32),
            pltpu.VMEM((16, 128), F32),
            pltpu.VMEM((16, 128), F32),
            pltpu.SemaphoreType.DMA,
            pltpu.SemaphoreType.DMA,
        ],
    )


_agg4 = _make_agg(4)
_agg2 = _make_agg(2)


def _tc_prep(degT, x, W1):
    def body(deg_ref, x_ref, w_ref, dinv_ref, g_ref):
        d = deg_ref[...]
        dinv = lax.rsqrt(d[:, 0:1] + d[:, 1:2] + 1.0)
        g = jnp.dot(x_ref[...], w_ref[...], preferred_element_type=F32) * dinv
        dinv_ref[...] = dinv
        for q in range(4):
            g_ref[q] = g[:, q * 128:(q + 1) * 128]

    outs = pl.pallas_call(
        body,
        grid=(N // BM,),
        in_specs=[
            pl.BlockSpec((BM, 2), lambda m: (m, 0)),
            pl.BlockSpec((BM, 256), lambda m: (m, 0)),
            pl.BlockSpec((256, 512), lambda m: (0, 0)),
        ],
        out_specs=[
            pl.BlockSpec((BM, 1), lambda m: (m, 0)),
            pl.BlockSpec((4, BM, 128), lambda m: (0, m, 0)),
        ],
        out_shape=[
            jax.ShapeDtypeStruct((N, 1), F32),
            jax.ShapeDtypeStruct((4, N, 128), F32),
        ],
    )(degT, x, W1)
    return outs[0], outs[1]


def _tc_mid(s_in, g_in, dinv, b2d, W, c_out):
    c_in = s_in.shape[0]

    def body(s_ref, gin_ref, dinv_ref, b_ref, w_ref, g_ref):
        dinv = dinv_ref[...]
        hs = [jax.nn.relu((s_ref[q] + gin_ref[q]) * dinv + b_ref[q])
              for q in range(c_in)]
        h = jnp.concatenate(hs, axis=1)
        g = jnp.dot(h, w_ref[...], preferred_element_type=F32) * dinv
        for q in range(c_out):
            g_ref[q] = g[:, q * 128:(q + 1) * 128]

    return pl.pallas_call(
        body,
        grid=(N // BM,),
        in_specs=[
            pl.BlockSpec((c_in, BM, 128), lambda m: (0, m, 0)),
            pl.BlockSpec((c_in, BM, 128), lambda m: (0, m, 0)),
            pl.BlockSpec((BM, 1), lambda m: (m, 0)),
            pl.BlockSpec((c_in, 128), lambda m: (0, 0)),
            pl.BlockSpec((128 * c_in, 128 * c_out), lambda m: (0, 0)),
        ],
        out_specs=pl.BlockSpec((c_out, BM, 128), lambda m: (0, m, 0)),
        out_shape=jax.ShapeDtypeStruct((c_out, N, 128), F32),
    )(s_in, g_in, dinv, b2d, W)


def _tc_final(s_in, g_in, dinv, b2d):
    def body(s_ref, gin_ref, dinv_ref, b_ref, out_ref):
        dinv = dinv_ref[...]
        out_ref[...] = jnp.concatenate(
            [(s_ref[q] + gin_ref[q]) * dinv + b_ref[q] for q in range(2)],
            axis=1)

    return pl.pallas_call(
        body,
        grid=(N // BM,),
        in_specs=[
            pl.BlockSpec((2, BM, 128), lambda m: (0, m, 0)),
            pl.BlockSpec((2, BM, 128), lambda m: (0, m, 0)),
            pl.BlockSpec((BM, 1), lambda m: (m, 0)),
            pl.BlockSpec((2, 128), lambda m: (0, 0)),
        ],
        out_specs=pl.BlockSpec((BM, 256), lambda m: (m, 0)),
        out_shape=jax.ShapeDtypeStruct((N, 256), F32),
    )(s_in, g_in, dinv, b2d)


def kernel(x, edge_index, W1, b1, W2, b2, W3, b3, W4, b4):
    ei = edge_index.astype(jnp.int32).reshape(2, ER, 128)
    pad = jnp.stack([jnp.zeros((30, 128), jnp.int32),
                     jnp.full((30, 128), N, jnp.int32)])
    e2d = jnp.concatenate([ei, pad], axis=1)
    deg0, deg1 = _deg(e2d)
    degT = jnp.stack([deg0, deg1], axis=1)
    dinv, g1 = _tc_prep(degT, x, W1)
    s1 = _agg4(e2d, g1)
    g2 = _tc_mid(s1, g1, dinv, b1.reshape(4, 128), W2, 4)
    s2 = _agg4(e2d, g2)
    g3 = _tc_mid(s2, g2, dinv, b2.reshape(4, 128), W3, 4)
    s3 = _agg4(e2d, g3)
    g4 = _tc_mid(s3, g3, dinv, b3.reshape(4, 128), W4, 2)
    s4 = _agg2(e2d, g4)
    return _tc_final(s4, g4, dinv, b4.reshape(2, 128))


# speedup vs baseline: 1.9382x; 1.9382x over previous
import functools

import jax
import jax.numpy as jnp
from jax import lax
from jax.experimental import pallas as pl
from jax.experimental.pallas import tpu as pltpu
from jax.experimental.pallas import tpu_sc as plsc

N = 10000
E = 160000
ER = E // 128
NC, NS = 2, 16
BM = 1000
F32 = jnp.float32


def _sc_mesh():
    return plsc.VectorSubcoreMesh(core_axis_name="c", subcore_axis_name="s")


@functools.partial(
    pl.kernel,
    out_type=[jax.ShapeDtypeStruct((N,), F32),
              jax.ShapeDtypeStruct((N,), F32)],
    mesh=_sc_mesh(),
    scratch_types=[
        pltpu.VMEM_SHARED((N + 8,), F32),
        pltpu.VMEM((1024,), F32),
        pltpu.VMEM((128,), F32),
        pltpu.VMEM((40, 128), jnp.int32),
        pltpu.VMEM((640,), F32),
    ],
)
def _deg(e2d_hbm, deg0_hbm, deg1_hbm, deg_sp, zbuf, ones, idxb, dbuf):
    c = lax.axis_index("c")
    s = lax.axis_index("s")
    zv = jnp.zeros((16,), F32)
    for off in range(0, 1024, 16):
        zbuf[pl.ds(off, 16)] = zv
    ov = jnp.ones((16,), F32)
    for off in range(0, 128, 16):
        ones[pl.ds(off, 16)] = ov

    @pl.when(s < 10)
    def _():
        pltpu.sync_copy(zbuf.at[pl.ds(0, 1000)],
                        deg_sp.at[pl.ds(s * 1000, 1000)])

    plsc.subcore_barrier()

    t = 16 * c + s
    pltpu.sync_copy(e2d_hbm.at[1, pl.ds(40 * t, 40)], idxb)

    def eb(j, carry):
        pltpu.sync_copy(ones, deg_sp.at[idxb.at[j]], add=True)
        return carry

    lax.fori_loop(0, jnp.where(t == 31, 10, 40), eb, 0)

    plsc.subcore_barrier()

    def writeout(dst):
        @pl.when(s < 15)
        def _():
            pltpu.sync_copy(deg_sp.at[pl.ds(s * 640, 640)], dbuf)
            pltpu.sync_copy(dbuf, dst.at[pl.ds(s * 640, 640)])

        @pl.when(s == 15)
        def _():
            pltpu.sync_copy(deg_sp.at[pl.ds(9600, 400)],
                            dbuf.at[pl.ds(0, 400)])
            pltpu.sync_copy(dbuf.at[pl.ds(0, 400)],
                            dst.at[pl.ds(9600, 400)])

    @pl.when(c == 0)
    def _():
        writeout(deg0_hbm)

    @pl.when(c == 1)
    def _():
        writeout(deg1_hbm)


def _make_agg(C):
    C2 = C // NC

    def body(e2d_hbm, g_hbm, s_hbm, acc, ridx, cidx, bufa, obuf, zbuf,
             sema, semb):
        c = lax.axis_index("c")
        s = lax.axis_index("s")

        zv = jnp.zeros((16,), F32)

        def zrow(r, carry):
            for off in range(0, 128, 16):
                zbuf[r, pl.ds(off, 16)] = zv
            return carry

        lax.fori_loop(0, 16, zrow, 0)

        nslabs = jnp.where(s == 0, 40, 39)
        nbase = 16 * (39 * s + jnp.minimum(s, 1))

        pltpu.sync_copy(e2d_hbm.at[0, pl.ds(s * 80, 80)], ridx)
        pltpu.sync_copy(e2d_hbm.at[1, pl.ds(s * 80, 80)], cidx)
        nblk = jnp.where(s == 15, 50, 80)

        def chunk_pass(qi, carry):
            q = c * C2 + qi
            g = g_hbm.at[q]
            so = s_hbm.at[q]

            def zi(i, cy):
                pltpu.sync_copy(zbuf, acc.at[pl.ds(nbase + 16 * i, 16)])
                return cy

            lax.fori_loop(0, nslabs, zi, 0)

            plsc.subcore_barrier()

            def eb(j, cy):
                pltpu.sync_copy(g.at[ridx.at[j]], bufa)
                pltpu.sync_copy(bufa, acc.at[cidx.at[j]], add=True)
                return cy

            lax.fori_loop(0, nblk, eb, 0)

            plsc.subcore_barrier()

            def wi(i, cy):
                base = nbase + 16 * i
                pltpu.sync_copy(acc.at[pl.ds(base, 16)], obuf)
                pltpu.sync_copy(obuf, so.at[pl.ds(base, 16)])
                return cy

            lax.fori_loop(0, nslabs, wi, 0)

            plsc.subcore_barrier()
            return carry

        lax.fori_loop(0, C2, chunk_pass, 0)

    return pl.kernel(
        body,
        out_type=jax.ShapeDtypeStruct((C, N, 128), F32),
        mesh=_sc_mesh(),
        scratch_types=[
            pltpu.VMEM_SHARED((N + 8, 128), F32),
            pltpu.VMEM((80, 128), jnp.int32),
            pltpu.VMEM((80, 128), jnp.int32),
            pltpu.VMEM((128, 128), F32),
            pltpu.VMEM((16, 128), F32),
            pltpu.VMEM((16, 128), F32),
            pltpu.SemaphoreType.DMA,
            pltpu.SemaphoreType.DMA,
        ],
    )


_agg4 = _make_agg(4)
_agg2 = _make_agg(2)


def _tc_prep(degT, x, W1):
    def body(deg_ref, x_ref, w_ref, dinv_ref, g_ref):
        d = deg_ref[...]
        dinv = lax.rsqrt(d[:, 0:1] + d[:, 1:2] + 1.0)
        g = jnp.dot(x_ref[...], w_ref[...], preferred_element_type=F32) * dinv
        dinv_ref[...] = dinv
        for q in range(4):
            g_ref[q] = g[:, q * 128:(q + 1) * 128]

    outs = pl.pallas_call(
        body,
        grid=(N // BM,),
        in_specs=[
            pl.BlockSpec((BM, 2), lambda m: (m, 0)),
            pl.BlockSpec((BM, 256), lambda m: (m, 0)),
            pl.BlockSpec((256, 512), lambda m: (0, 0)),
        ],
        out_specs=[
            pl.BlockSpec((BM, 1), lambda m: (m, 0)),
            pl.BlockSpec((4, BM, 128), lambda m: (0, m, 0)),
        ],
        out_shape=[
            jax.ShapeDtypeStruct((N, 1), F32),
            jax.ShapeDtypeStruct((4, N, 128), F32),
        ],
    )(degT, x, W1)
    return outs[0], outs[1]


def _tc_mid(s_in, g_in, dinv, b2d, W, c_out):
    c_in = s_in.shape[0]

    def body(s_ref, gin_ref, dinv_ref, b_ref, w_ref, g_ref):
        dinv = dinv_ref[...]
        hs = [jax.nn.relu((s_ref[q] + gin_ref[q]) * dinv + b_ref[q])
              for q in range(c_in)]
        h = jnp.concatenate(hs, axis=1)
        g = jnp.dot(h, w_ref[...], preferred_element_type=F32) * dinv
        for q in range(c_out):
            g_ref[q] = g[:, q * 128:(q + 1) * 128]

    return pl.pallas_call(
        body,
        grid=(N // BM,),
        in_specs=[
            pl.BlockSpec((c_in, BM, 128), lambda m: (0, m, 0)),
            pl.BlockSpec((c_in, BM, 128), lambda m: (0, m, 0)),
            pl.BlockSpec((BM, 1), lambda m: (m, 0)),
            pl.BlockSpec((c_in, 128), lambda m: (0, 0)),
            pl.BlockSpec((128 * c_in, 128 * c_out), lambda m: (0, 0)),
        ],
        out_specs=pl.BlockSpec((c_out, BM, 128), lambda m: (0, m, 0)),
        out_shape=jax.ShapeDtypeStruct((c_out, N, 128), F32),
    )(s_in, g_in, dinv, b2d, W)


def _tc_final(s_in, g_in, dinv, b2d):
    def body(s_ref, gin_ref, dinv_ref, b_ref, out_ref):
        dinv = dinv_ref[...]
        out_ref[...] = jnp.concatenate(
            [(s_ref[q] + gin_ref[q]) * dinv + b_ref[q] for q in range(2)],
            axis=1)

    return pl.pallas_call(
        body,
        grid=(N // BM,),
        in_specs=[
            pl.BlockSpec((2, BM, 128), lambda m: (0, m, 0)),
            pl.BlockSpec((2, BM, 128), lambda m: (0, m, 0)),
            pl.BlockSpec((BM, 1), lambda m: (m, 0)),
            pl.BlockSpec((2, 128), lambda m: (0, 0)),
        ],
        out_specs=pl.BlockSpec((BM, 256), lambda m: (m, 0)),
        out_shape=jax.ShapeDtypeStruct((N, 256), F32),
    )(s_in, g_in, dinv, b2d)


def kernel(x, edge_index, W1, b1, W2, b2, W3, b3, W4, b4):
    ei = edge_index.astype(jnp.int32).reshape(2, ER, 128)
    pad = jnp.stack([jnp.zeros((30, 128), jnp.int32),
                     jnp.full((30, 128), N, jnp.int32)])
    e2d = jnp.concatenate([ei, pad], axis=1)
    deg0, deg1 = _deg(e2d)
    degT = jnp.stack([deg0, deg1], axis=1)
    dinv, g1 = _tc_prep(degT, x, W1)
    s1 = _agg4(e2d, g1)
    g2 = _tc_mid(s1, g1, dinv, b1.reshape(4, 128), W2, 4)
    s2 = _agg4(e2d, g2)
    g3 = _tc_mid(s2, g2, dinv, b2.reshape(4, 128), W3, 4)
    s3 = _agg4(e2d, g3)
    g4 = _tc_mid(s3, g3, dinv, b3.reshape(4, 128), W4, 2)
    s4 = _agg2(e2d, g4)
    return _tc_final(s4, g4, dinv, b4.reshape(2, 128))


# speedup vs baseline: 1.9407x; 1.0013x over previous
import functools

import jax
import jax.numpy as jnp
from jax import lax
from jax.experimental import pallas as pl
from jax.experimental.pallas import tpu as pltpu
from jax.experimental.pallas import tpu_sc as plsc

N = 10000
E = 160000
ER = E // 128
NC, NS = 2, 16
BM = 1000
F32 = jnp.float32


def _sc_mesh():
    return plsc.VectorSubcoreMesh(core_axis_name="c", subcore_axis_name="s")


@functools.partial(
    pl.kernel,
    out_type=[jax.ShapeDtypeStruct((N,), F32),
              jax.ShapeDtypeStruct((N,), F32)],
    mesh=_sc_mesh(),
    scratch_types=[
        pltpu.VMEM_SHARED((N + 8,), F32),
        pltpu.VMEM((1024,), F32),
        pltpu.VMEM((128,), F32),
        pltpu.VMEM((40, 128), jnp.int32),
        pltpu.VMEM((640,), F32),
    ],
)
def _deg(e2d_hbm, deg0_hbm, deg1_hbm, deg_sp, zbuf, ones, idxb, dbuf):
    c = lax.axis_index("c")
    s = lax.axis_index("s")
    zv = jnp.zeros((16,), F32)
    for off in range(0, 1024, 16):
        zbuf[pl.ds(off, 16)] = zv
    ov = jnp.ones((16,), F32)
    for off in range(0, 128, 16):
        ones[pl.ds(off, 16)] = ov

    @pl.when(s < 10)
    def _():
        pltpu.sync_copy(zbuf.at[pl.ds(0, 1000)],
                        deg_sp.at[pl.ds(s * 1000, 1000)])

    plsc.subcore_barrier()

    t = 16 * c + s
    pltpu.sync_copy(e2d_hbm.at[1, pl.ds(40 * t, 40)], idxb)

    def eb(j, carry):
        pltpu.sync_copy(ones, deg_sp.at[idxb.at[j]], add=True)
        return carry

    lax.fori_loop(0, jnp.where(t == 31, 10, 40), eb, 0)

    plsc.subcore_barrier()

    def writeout(dst):
        @pl.when(s < 15)
        def _():
            pltpu.sync_copy(deg_sp.at[pl.ds(s * 640, 640)], dbuf)
            pltpu.sync_copy(dbuf, dst.at[pl.ds(s * 640, 640)])

        @pl.when(s == 15)
        def _():
            pltpu.sync_copy(deg_sp.at[pl.ds(9600, 400)],
                            dbuf.at[pl.ds(0, 400)])
            pltpu.sync_copy(dbuf.at[pl.ds(0, 400)],
                            dst.at[pl.ds(9600, 400)])

    @pl.when(c == 0)
    def _():
        writeout(deg0_hbm)

    @pl.when(c == 1)
    def _():
        writeout(deg1_hbm)


def _make_agg(C):
    C2 = C // NC

    def body(e2d_hbm, g_hbm, s_hbm, acc, ridx, cidx, bufa, obuf, zbuf):
        c = lax.axis_index("c")
        s = lax.axis_index("s")

        zv = jnp.zeros((16,), F32)

        def zrow(r, carry):
            for off in range(0, 128, 16):
                zbuf[r, pl.ds(off, 16)] = zv
            return carry

        lax.fori_loop(0, 16, zrow, 0)

        nslabs = jnp.where(s == 0, 40, 39)
        nbase = 16 * (39 * s + jnp.minimum(s, 1))

        pltpu.sync_copy(e2d_hbm.at[0, pl.ds(s * 80, 80)], ridx)
        pltpu.sync_copy(e2d_hbm.at[1, pl.ds(s * 80, 80)], cidx)
        nblk = jnp.where(s == 15, 50, 80)

        def chunk_pass(qi, carry):
            q = c * C2 + qi
            g = g_hbm.at[q]
            so = s_hbm.at[q]

            def zi(i, cy):
                pltpu.sync_copy(zbuf, acc.at[pl.ds(nbase + 16 * i, 16)])
                return cy

            lax.fori_loop(0, nslabs, zi, 0)

            plsc.subcore_barrier()

            def eb(j, cy):
                pltpu.sync_copy(g.at[ridx.at[j]], bufa)
                pltpu.sync_copy(bufa, acc.at[cidx.at[j]], add=True)
                return cy

            lax.fori_loop(0, nblk, eb, 0)

            plsc.subcore_barrier()

            def wi(i, cy):
                base = nbase + 16 * i
                pltpu.sync_copy(acc.at[pl.ds(base, 16)], obuf)
                pltpu.sync_copy(obuf, so.at[pl.ds(base, 16)])
                return cy

            lax.fori_loop(0, nslabs, wi, 0)

            plsc.subcore_barrier()
            return carry

        lax.fori_loop(0, C2, chunk_pass, 0)

    return pl.kernel(
        body,
        out_type=jax.ShapeDtypeStruct((C, N, 128), F32),
        mesh=_sc_mesh(),
        scratch_types=[
            pltpu.VMEM_SHARED((N + 8, 128), F32),
            pltpu.VMEM((80, 128), jnp.int32),
            pltpu.VMEM((80, 128), jnp.int32),
            pltpu.VMEM((128, 128), F32),
            pltpu.VMEM((16, 128), F32),
            pltpu.VMEM((16, 128), F32),
        ],
    )


_agg4 = _make_agg(4)
_agg2 = _make_agg(2)


def _tc_prep(degT, x, W1):
    def body(deg_ref, x_ref, w_ref, dinv_ref, g_ref):
        d = deg_ref[...]
        dinv = lax.rsqrt(d[:, 0:1] + d[:, 1:2] + 1.0)
        g = jnp.dot(x_ref[...], w_ref[...], preferred_element_type=F32) * dinv
        dinv_ref[...] = dinv
        for q in range(4):
            g_ref[q] = g[:, q * 128:(q + 1) * 128]

    outs = pl.pallas_call(
        body,
        grid=(N // BM,),
        in_specs=[
            pl.BlockSpec((BM, 2), lambda m: (m, 0)),
            pl.BlockSpec((BM, 256), lambda m: (m, 0)),
            pl.BlockSpec((256, 512), lambda m: (0, 0)),
        ],
        out_specs=[
            pl.BlockSpec((BM, 1), lambda m: (m, 0)),
            pl.BlockSpec((4, BM, 128), lambda m: (0, m, 0)),
        ],
        out_shape=[
            jax.ShapeDtypeStruct((N, 1), F32),
            jax.ShapeDtypeStruct((4, N, 128), F32),
        ],
    )(degT, x, W1)
    return outs[0], outs[1]


def _tc_mid(s_in, g_in, dinv, b2d, W, c_out):
    c_in = s_in.shape[0]

    def body(s_ref, gin_ref, dinv_ref, b_ref, w_ref, g_ref):
        dinv = dinv_ref[...]
        hs = [jax.nn.relu((s_ref[q] + gin_ref[q]) * dinv + b_ref[q])
              for q in range(c_in)]
        h = jnp.concatenate(hs, axis=1)
        g = jnp.dot(h, w_ref[...], preferred_element_type=F32) * dinv
        for q in range(c_out):
            g_ref[q] = g[:, q * 128:(q + 1) * 128]

    return pl.pallas_call(
        body,
        grid=(N // BM,),
        in_specs=[
            pl.BlockSpec((c_in, BM, 128), lambda m: (0, m, 0)),
            pl.BlockSpec((c_in, BM, 128), lambda m: (0, m, 0)),
            pl.BlockSpec((BM, 1), lambda m: (m, 0)),
            pl.BlockSpec((c_in, 128), lambda m: (0, 0)),
            pl.BlockSpec((128 * c_in, 128 * c_out), lambda m: (0, 0)),
        ],
        out_specs=pl.BlockSpec((c_out, BM, 128), lambda m: (0, m, 0)),
        out_shape=jax.ShapeDtypeStruct((c_out, N, 128), F32),
    )(s_in, g_in, dinv, b2d, W)


def _tc_final(s_in, g_in, dinv, b2d):
    def body(s_ref, gin_ref, dinv_ref, b_ref, out_ref):
        dinv = dinv_ref[...]
        out_ref[...] = jnp.concatenate(
            [(s_ref[q] + gin_ref[q]) * dinv + b_ref[q] for q in range(2)],
            axis=1)

    return pl.pallas_call(
        body,
        grid=(N // BM,),
        in_specs=[
            pl.BlockSpec((2, BM, 128), lambda m: (0, m, 0)),
            pl.BlockSpec((2, BM, 128), lambda m: (0, m, 0)),
            pl.BlockSpec((BM, 1), lambda m: (m, 0)),
            pl.BlockSpec((2, 128), lambda m: (0, 0)),
        ],
        out_specs=pl.BlockSpec((BM, 256), lambda m: (m, 0)),
        out_shape=jax.ShapeDtypeStruct((N, 256), F32),
    )(s_in, g_in, dinv, b2d)


def kernel(x, edge_index, W1, b1, W2, b2, W3, b3, W4, b4):
    ei = edge_index.astype(jnp.int32).reshape(2, ER, 128)
    pad = jnp.stack([jnp.zeros((30, 128), jnp.int32),
                     jnp.full((30, 128), N, jnp.int32)])
    e2d = jnp.concatenate([ei, pad], axis=1)
    deg0, deg1 = _deg(e2d)
    degT = jnp.stack([deg0, deg1], axis=1)
    dinv, g1 = _tc_prep(degT, x, W1)
    s1 = _agg4(e2d, g1)
    g2 = _tc_mid(s1, g1, dinv, b1.reshape(4, 128), W2, 4)
    s2 = _agg4(e2d, g2)
    g3 = _tc_mid(s2, g2, dinv, b2.reshape(4, 128), W3, 4)
    s3 = _agg4(e2d, g3)
    g4 = _tc_mid(s3, g3, dinv, b3.reshape(4, 128), W4, 2)
    s4 = _agg2(e2d, g4)
    return _tc_final(s4, g4, dinv, b4.reshape(2, 128))


# speedup vs baseline: 2.2070x; 1.1372x over previous
import functools

import jax
import jax.numpy as jnp
from jax import lax
from jax.experimental import pallas as pl
from jax.experimental.pallas import tpu as pltpu
from jax.experimental.pallas import tpu_sc as plsc

N = 10000
E = 160000
ER = E // 128
NC, NS = 2, 16
BM = 1000
F32 = jnp.float32


def _sc_mesh():
    return plsc.VectorSubcoreMesh(core_axis_name="c", subcore_axis_name="s")


@functools.partial(
    pl.kernel,
    out_type=[jax.ShapeDtypeStruct((N,), F32),
              jax.ShapeDtypeStruct((N,), F32)],
    mesh=_sc_mesh(),
    scratch_types=[
        pltpu.VMEM_SHARED((N + 8,), F32),
        pltpu.VMEM((1024,), F32),
        pltpu.VMEM((128,), F32),
        pltpu.VMEM((40, 128), jnp.int32),
        pltpu.VMEM((640,), F32),
    ],
)
def _deg(e2d_hbm, deg0_hbm, deg1_hbm, deg_sp, zbuf, ones, idxb, dbuf):
    c = lax.axis_index("c")
    s = lax.axis_index("s")
    zv = jnp.zeros((16,), F32)
    for off in range(0, 1024, 16):
        zbuf[pl.ds(off, 16)] = zv
    ov = jnp.ones((16,), F32)
    for off in range(0, 128, 16):
        ones[pl.ds(off, 16)] = ov

    @pl.when(s < 10)
    def _():
        pltpu.sync_copy(zbuf.at[pl.ds(0, 1000)],
                        deg_sp.at[pl.ds(s * 1000, 1000)])

    plsc.subcore_barrier()

    t = 16 * c + s
    pltpu.sync_copy(e2d_hbm.at[1, pl.ds(40 * t, 40)], idxb)

    def eb(j, carry):
        pltpu.sync_copy(ones, deg_sp.at[idxb.at[j]], add=True)
        return carry

    lax.fori_loop(0, jnp.where(t == 31, 10, 40), eb, 0)

    plsc.subcore_barrier()

    def writeout(dst):
        @pl.when(s < 15)
        def _():
            pltpu.sync_copy(deg_sp.at[pl.ds(s * 640, 640)], dbuf)
            pltpu.sync_copy(dbuf, dst.at[pl.ds(s * 640, 640)])

        @pl.when(s == 15)
        def _():
            pltpu.sync_copy(deg_sp.at[pl.ds(9600, 400)],
                            dbuf.at[pl.ds(0, 400)])
            pltpu.sync_copy(dbuf.at[pl.ds(0, 400)],
                            dst.at[pl.ds(9600, 400)])

    @pl.when(c == 0)
    def _():
        writeout(deg0_hbm)

    @pl.when(c == 1)
    def _():
        writeout(deg1_hbm)


def _make_agg(C):
    C2 = C // NC

    def body(e2d_hbm, g_hbm, s_hbm, acc, ridx, cidx, bufa, bufb, obuf,
             zbuf, sema, semb):
        c = lax.axis_index("c")
        s = lax.axis_index("s")

        zv = jnp.zeros((16,), F32)

        def zrow(r, carry):
            for off in range(0, 128, 16):
                zbuf[r, pl.ds(off, 16)] = zv
            return carry

        lax.fori_loop(0, 16, zrow, 0)

        nslabs = jnp.where(s == 0, 40, 39)
        nbase = 16 * (39 * s + jnp.minimum(s, 1))


        def chunk_pass(qi, carry):
            q = c * C2 + qi
            g = g_hbm.at[q]
            so = s_hbm.at[q]

            def zi(i, cy):
                pltpu.sync_copy(zbuf, acc.at[pl.ds(nbase + 16 * i, 16)])
                return cy

            lax.fori_loop(0, nslabs, zi, 0)

            plsc.subcore_barrier()

            def half(h, cy):
                base = s * 80 + 40 * h
                pltpu.sync_copy(e2d_hbm.at[0, pl.ds(base, 40)], ridx)
                pltpu.sync_copy(e2d_hbm.at[1, pl.ds(base, 40)], cidx)
                nh = jnp.where(s == 15, jnp.where(h == 0, 40, 10), 40)

                def eb2(p, cy2):
                    j = 2 * p
                    ha = pltpu.async_copy(g.at[ridx.at[j]], bufa, sema)
                    hb = pltpu.async_copy(g.at[ridx.at[j + 1]], bufb, semb)
                    ha.wait()
                    pltpu.sync_copy(bufa, acc.at[cidx.at[j]], add=True)
                    hb.wait()
                    pltpu.sync_copy(bufb, acc.at[cidx.at[j + 1]], add=True)
                    return cy2

                lax.fori_loop(0, nh // 2, eb2, 0)
                return cy

            lax.fori_loop(0, 2, half, 0)

            plsc.subcore_barrier()

            def wi(i, cy):
                base = nbase + 16 * i
                pltpu.sync_copy(acc.at[pl.ds(base, 16)], obuf)
                pltpu.sync_copy(obuf, so.at[pl.ds(base, 16)])
                return cy

            lax.fori_loop(0, nslabs, wi, 0)

            plsc.subcore_barrier()
            return carry

        lax.fori_loop(0, C2, chunk_pass, 0)

    return pl.kernel(
        body,
        out_type=jax.ShapeDtypeStruct((C, N, 128), F32),
        mesh=_sc_mesh(),
        scratch_types=[
            pltpu.VMEM_SHARED((N + 8, 128), F32),
            pltpu.VMEM((40, 128), jnp.int32),
            pltpu.VMEM((40, 128), jnp.int32),
            pltpu.VMEM((128, 128), F32),
            pltpu.VMEM((128, 128), F32),
            pltpu.VMEM((16, 128), F32),
            pltpu.VMEM((16, 128), F32),
            pltpu.SemaphoreType.DMA,
            pltpu.SemaphoreType.DMA,
        ],
    )


_agg4 = _make_agg(4)
_agg2 = _make_agg(2)


def _tc_prep(degT, x, W1):
    def body(deg_ref, x_ref, w_ref, dinv_ref, g_ref):
        d = deg_ref[...]
        dinv = lax.rsqrt(d[:, 0:1] + d[:, 1:2] + 1.0)
        g = jnp.dot(x_ref[...], w_ref[...], preferred_element_type=F32) * dinv
        dinv_ref[...] = dinv
        for q in range(4):
            g_ref[q] = g[:, q * 128:(q + 1) * 128]

    outs = pl.pallas_call(
        body,
        grid=(N // BM,),
        in_specs=[
            pl.BlockSpec((BM, 2), lambda m: (m, 0)),
            pl.BlockSpec((BM, 256), lambda m: (m, 0)),
            pl.BlockSpec((256, 512), lambda m: (0, 0)),
        ],
        out_specs=[
            pl.BlockSpec((BM, 1), lambda m: (m, 0)),
            pl.BlockSpec((4, BM, 128), lambda m: (0, m, 0)),
        ],
        out_shape=[
            jax.ShapeDtypeStruct((N, 1), F32),
            jax.ShapeDtypeStruct((4, N, 128), F32),
        ],
    )(degT, x, W1)
    return outs[0], outs[1]


def _tc_mid(s_in, g_in, dinv, b2d, W, c_out):
    c_in = s_in.shape[0]

    def body(s_ref, gin_ref, dinv_ref, b_ref, w_ref, g_ref):
        dinv = dinv_ref[...]
        hs = [jax.nn.relu((s_ref[q] + gin_ref[q]) * dinv + b_ref[q])
              for q in range(c_in)]
        h = jnp.concatenate(hs, axis=1)
        g = jnp.dot(h, w_ref[...], preferred_element_type=F32) * dinv
        for q in range(c_out):
            g_ref[q] = g[:, q * 128:(q + 1) * 128]

    return pl.pallas_call(
        body,
        grid=(N // BM,),
        in_specs=[
            pl.BlockSpec((c_in, BM, 128), lambda m: (0, m, 0)),
            pl.BlockSpec((c_in, BM, 128), lambda m: (0, m, 0)),
            pl.BlockSpec((BM, 1), lambda m: (m, 0)),
            pl.BlockSpec((c_in, 128), lambda m: (0, 0)),
            pl.BlockSpec((128 * c_in, 128 * c_out), lambda m: (0, 0)),
        ],
        out_specs=pl.BlockSpec((c_out, BM, 128), lambda m: (0, m, 0)),
        out_shape=jax.ShapeDtypeStruct((c_out, N, 128), F32),
    )(s_in, g_in, dinv, b2d, W)


def _tc_final(s_in, g_in, dinv, b2d):
    def body(s_ref, gin_ref, dinv_ref, b_ref, out_ref):
        dinv = dinv_ref[...]
        out_ref[...] = jnp.concatenate(
            [(s_ref[q] + gin_ref[q]) * dinv + b_ref[q] for q in range(2)],
            axis=1)

    return pl.pallas_call(
        body,
        grid=(N // BM,),
        in_specs=[
            pl.BlockSpec((2, BM, 128), lambda m: (0, m, 0)),
            pl.BlockSpec((2, BM, 128), lambda m: (0, m, 0)),
            pl.BlockSpec((BM, 1), lambda m: (m, 0)),
            pl.BlockSpec((2, 128), lambda m: (0, 0)),
        ],
        out_specs=pl.BlockSpec((BM, 256), lambda m: (m, 0)),
        out_shape=jax.ShapeDtypeStruct((N, 256), F32),
    )(s_in, g_in, dinv, b2d)


def kernel(x, edge_index, W1, b1, W2, b2, W3, b3, W4, b4):
    ei = edge_index.astype(jnp.int32).reshape(2, ER, 128)
    pad = jnp.stack([jnp.zeros((30, 128), jnp.int32),
                     jnp.full((30, 128), N, jnp.int32)])
    e2d = jnp.concatenate([ei, pad], axis=1)
    deg0, deg1 = _deg(e2d)
    degT = jnp.stack([deg0, deg1], axis=1)
    dinv, g1 = _tc_prep(degT, x, W1)
    s1 = _agg4(e2d, g1)
    g2 = _tc_mid(s1, g1, dinv, b1.reshape(4, 128), W2, 4)
    s2 = _agg4(e2d, g2)
    g3 = _tc_mid(s2, g2, dinv, b2.reshape(4, 128), W3, 4)
    s3 = _agg4(e2d, g3)
    g4 = _tc_mid(s3, g3, dinv, b3.reshape(4, 128), W4, 2)
    s4 = _agg2(e2d, g4)
    return _tc_final(s4, g4, dinv, b4.reshape(2, 128))
